# baseline (speedup 1.0000x reference)
import jax
import jax.numpy as jnp
from jax import lax
from jax.experimental import pallas as pl
from jax.experimental.pallas import tpu as pltpu
from jax.experimental.pallas import tpu_sc as plsc

B = 16384
OBS_DIM = 512
HID = 256
CODE_DIM = 64
K = 1024

TB = 2048
N_STEPS = B // TB

GATHER_WINDOW = 128


def _tc_body(obs_ref, w1_ref, b1_ref, w2_ref, b2_ref, cbt2_ref,
             idx_ref, loss_ref, c2_ref):
    i = pl.program_id(0)
    bf = jnp.bfloat16

    @pl.when(i == 0)
    def _():
        cbt2 = cbt2_ref[...]
        c2_ref[...] = 0.25 * jnp.sum(cbt2 * cbt2, axis=0, keepdims=True)

    obs = obs_ref[...].astype(bf)
    h = jax.nn.relu(
        lax.dot(obs, w1_ref[...].astype(bf),
                preferred_element_type=jnp.float32) + b1_ref[...])
    f = lax.dot(h.astype(bf), w2_ref[...].astype(bf),
                preferred_element_type=jnp.float32) + b2_ref[...]
    s2 = lax.dot(f.astype(bf), cbt2_ref[...].astype(bf),
                 preferred_element_type=jnp.float32)
    f2 = jnp.sum(f * f, axis=1, keepdims=True)
    d = (f2 + s2) + c2_ref[...]
    m = jnp.min(d, axis=1, keepdims=True)
    iota = lax.broadcasted_iota(jnp.int32, (TB, K), 1)
    idx = jnp.min(jnp.where(d == m, iota, K), axis=1, keepdims=True)
    idx_ref[...] = jnp.minimum(idx, K - 1)
    part = jnp.sum(m)[None, None]
    acc = jnp.where(i == 0, part, loss_ref[...] + part)
    loss_ref[...] = jnp.where(i == N_STEPS - 1,
                              acc * (1.25 / (B * CODE_DIM)), acc)


def _tc_encode(observation, W1, b1r, W2, b2r, cbt2):
    return pl.pallas_call(
        _tc_body,
        grid=(N_STEPS,),
        in_specs=[
            pl.BlockSpec((TB, OBS_DIM), lambda i: (i, 0)),
            pl.BlockSpec((OBS_DIM, HID), lambda i: (0, 0)),
            pl.BlockSpec((1, HID), lambda i: (0, 0)),
            pl.BlockSpec((HID, CODE_DIM), lambda i: (0, 0)),
            pl.BlockSpec((1, CODE_DIM), lambda i: (0, 0)),
            pl.BlockSpec((CODE_DIM, K), lambda i: (0, 0)),
        ],
        out_specs=[
            pl.BlockSpec((TB, 1), lambda i: (i, 0)),
            pl.BlockSpec((1, 1), lambda i: (0, 0)),
        ],
        out_shape=[
            jax.ShapeDtypeStruct((B, 1), jnp.int32),
            jax.ShapeDtypeStruct((1, 1), jnp.float32),
        ],
        scratch_shapes=[pltpu.VMEM((1, K), jnp.float32)],
    )(observation, W1, b1r, W2, b2r, cbt2)


def _sc_gather(cb_pad, idx_row):
    @pl.kernel(
        out_type=jax.ShapeDtypeStruct((B, 128), cb_pad.dtype),
        mesh=plsc.VectorSubcoreMesh(core_axis_name="core",
                                    subcore_axis_name="subcore"),
    )
    def gather_kernel(cb_hbm, i_hbm, o_hbm):
        def body(i_vmem, o_vmem):
            pltpu.sync_copy(cb_hbm.at[i_vmem.at[0]], o_vmem)

        pltpu.emit_pipeline(
            body,
            grid=(B // GATHER_WINDOW,),
            in_specs=[pl.BlockSpec((1, GATHER_WINDOW),
                                   index_map=lambda i: (0, i))],
            out_specs=[pl.BlockSpec((GATHER_WINDOW, 128),
                                    index_map=lambda i: (i, 0))],
            core_axis_name=("core", "subcore"),
            dimension_semantics=(pltpu.PARALLEL,),
        )(i_hbm, o_hbm)

    return gather_kernel(cb_pad, idx_row)


@jax.jit
def kernel(observation, W1, b1, W2, b2, codebook):
    b1r = b1.reshape(1, HID)
    b2r = b2.reshape(1, CODE_DIM)
    cbt2 = -2.0 * codebook.T
    cb_pad = jnp.pad(codebook, ((0, 0), (0, 128 - CODE_DIM)))
    idx, loss = _tc_encode(observation, W1, b1r, W2, b2r, cbt2)
    gathered = _sc_gather(cb_pad, idx.reshape(1, B))
    return gathered[:, :CODE_DIM], loss.reshape(())


# speedup vs baseline: 1.0041x; 1.0041x over previous
import jax
import jax.numpy as jnp
from jax import lax
from jax.experimental import pallas as pl
from jax.experimental.pallas import tpu as pltpu
from jax.experimental.pallas import tpu_sc as plsc

B = 16384
OBS_DIM = 512
HID = 256
CODE_DIM = 64
K = 1024

TB = 2048
N_STEPS = B // TB

GATHER_WINDOW = 256


def _tc_body(obs_ref, w1_ref, b1_ref, w2_ref, b2_ref, cbt2_ref,
             idx_ref, loss_ref, c2_ref):
    i = pl.program_id(0)
    bf = jnp.bfloat16

    @pl.when(i == 0)
    def _():
        cbt2 = cbt2_ref[...]
        c2_ref[...] = 0.25 * jnp.sum(cbt2 * cbt2, axis=0, keepdims=True)

    obs = obs_ref[...].astype(bf)
    h = jax.nn.relu(
        lax.dot(obs, w1_ref[...].astype(bf),
                preferred_element_type=jnp.float32) + b1_ref[...])
    f = lax.dot(h.astype(bf), w2_ref[...].astype(bf),
                preferred_element_type=jnp.float32) + b2_ref[...]
    s2 = lax.dot(f.astype(bf), cbt2_ref[...].astype(bf),
                 preferred_element_type=jnp.float32)
    f2 = jnp.sum(f * f, axis=1, keepdims=True)
    d = (f2 + s2) + c2_ref[...]
    m = jnp.min(d, axis=1, keepdims=True)
    iota = lax.broadcasted_iota(jnp.int32, (TB, K), 1)
    idx = jnp.min(jnp.where(d == m, iota, K), axis=1, keepdims=True)
    idx_ref[...] = jnp.minimum(idx, K - 1)
    part = jnp.sum(m)[None, None]
    acc = jnp.where(i == 0, part, loss_ref[...] + part)
    loss_ref[...] = jnp.where(i == N_STEPS - 1,
                              acc * (1.25 / (B * CODE_DIM)), acc)


def _tc_encode(observation, W1, b1r, W2, b2r, cbt2):
    return pl.pallas_call(
        _tc_body,
        grid=(N_STEPS,),
        in_specs=[
            pl.BlockSpec((TB, OBS_DIM), lambda i: (i, 0)),
            pl.BlockSpec((OBS_DIM, HID), lambda i: (0, 0)),
            pl.BlockSpec((1, HID), lambda i: (0, 0)),
            pl.BlockSpec((HID, CODE_DIM), lambda i: (0, 0)),
            pl.BlockSpec((1, CODE_DIM), lambda i: (0, 0)),
            pl.BlockSpec((CODE_DIM, K), lambda i: (0, 0)),
        ],
        out_specs=[
            pl.BlockSpec((TB, 1), lambda i: (i, 0)),
            pl.BlockSpec((1, 1), lambda i: (0, 0)),
        ],
        out_shape=[
            jax.ShapeDtypeStruct((B, 1), jnp.int32),
            jax.ShapeDtypeStruct((1, 1), jnp.float32),
        ],
        scratch_shapes=[pltpu.VMEM((1, K), jnp.float32)],
    )(observation, W1, b1r, W2, b2r, cbt2)


def _sc_gather(cb_pad, idx_row):
    @pl.kernel(
        out_type=jax.ShapeDtypeStruct((B, 128), cb_pad.dtype),
        mesh=plsc.VectorSubcoreMesh(core_axis_name="core",
                                    subcore_axis_name="subcore"),
    )
    def gather_kernel(cb_hbm, i_hbm, o_hbm):
        def body(i_vmem, o_vmem):
            pltpu.sync_copy(cb_hbm.at[i_vmem.at[0]], o_vmem)

        pltpu.emit_pipeline(
            body,
            grid=(B // GATHER_WINDOW,),
            in_specs=[pl.BlockSpec((1, GATHER_WINDOW),
                                   index_map=lambda i: (0, i))],
            out_specs=[pl.BlockSpec((GATHER_WINDOW, 128),
                                    index_map=lambda i: (i, 0))],
            core_axis_name=("core", "subcore"),
            dimension_semantics=(pltpu.PARALLEL,),
        )(i_hbm, o_hbm)

    return gather_kernel(cb_pad, idx_row)


@jax.jit
def kernel(observation, W1, b1, W2, b2, codebook):
    b1r = b1.reshape(1, HID)
    b2r = b2.reshape(1, CODE_DIM)
    cbt2 = -2.0 * codebook.T
    cb_pad = jnp.pad(codebook, ((0, 0), (0, 128 - CODE_DIM)))
    idx, loss = _tc_encode(observation, W1, b1r, W2, b2r, cbt2)
    gathered = _sc_gather(cb_pad, idx.reshape(1, B))
    return gathered[:, :CODE_DIM], loss.reshape(())


# speedup vs baseline: 1.7439x; 1.7369x over previous
import jax
import jax.numpy as jnp
from jax import lax
from jax.experimental import pallas as pl
from jax.experimental.pallas import tpu as pltpu

B = 16384
OBS_DIM = 512
HID = 256
CODE_DIM = 64
K = 1024

TB = 2048
N_STEPS = B // TB


def _body(obs_ref, w1_ref, b1_ref, w2_ref, b2_ref, cbh_ref, cbt_ref,
          q_ref, loss_ref, c2_ref):
    i = pl.program_id(0)
    bf = jnp.bfloat16

    @pl.when(i == 0)
    def _():
        cbt = cbt_ref[...]
        c2_ref[...] = jnp.sum(cbt * cbt, axis=0, keepdims=True)

    obs = obs_ref[...].astype(bf)
    h = jax.nn.relu(
        lax.dot(obs, w1_ref[...].astype(bf),
                preferred_element_type=jnp.float32) + b1_ref[...])
    f = lax.dot(h.astype(bf), w2_ref[...].astype(bf),
                preferred_element_type=jnp.float32) + b2_ref[...]
    s = lax.dot(f.astype(bf), cbt_ref[...].astype(bf),
                preferred_element_type=jnp.float32)
    f2 = jnp.sum(f * f, axis=1, keepdims=True)
    d = (f2 - 2.0 * s) + c2_ref[...]
    m = jnp.min(d, axis=1, keepdims=True)
    iota = lax.broadcasted_iota(jnp.int32, (TB, K), 1)
    idx = jnp.min(jnp.where(d == m, iota, K), axis=1, keepdims=True)
    onehot = (iota == idx).astype(bf)
    q_ref[...] = lax.dot(onehot, cbh_ref[...], preferred_element_type=jnp.float32)
    part = jnp.sum(m)[None, None]
    acc = jnp.where(i == 0, part, loss_ref[...] + part)
    loss_ref[...] = jnp.where(i == N_STEPS - 1,
                              acc * (1.25 / (B * CODE_DIM)), acc)


@jax.jit
def kernel(observation, W1, b1, W2, b2, codebook):
    b1r = b1.reshape(1, HID)
    b2r = b2.reshape(1, CODE_DIM)
    cbt = codebook.T
    cb_hi = codebook.astype(jnp.bfloat16)
    quantized, loss = pl.pallas_call(
        _body,
        grid=(N_STEPS,),
        in_specs=[
            pl.BlockSpec((TB, OBS_DIM), lambda i: (i, 0)),
            pl.BlockSpec((OBS_DIM, HID), lambda i: (0, 0)),
            pl.BlockSpec((1, HID), lambda i: (0, 0)),
            pl.BlockSpec((HID, CODE_DIM), lambda i: (0, 0)),
            pl.BlockSpec((1, CODE_DIM), lambda i: (0, 0)),
            pl.BlockSpec((K, CODE_DIM), lambda i: (0, 0)),
            pl.BlockSpec((CODE_DIM, K), lambda i: (0, 0)),
        ],
        out_specs=[
            pl.BlockSpec((TB, CODE_DIM), lambda i: (i, 0)),
            pl.BlockSpec((1, 1), lambda i: (0, 0)),
        ],
        out_shape=[
            jax.ShapeDtypeStruct((B, CODE_DIM), jnp.float32),
            jax.ShapeDtypeStruct((1, 1), jnp.float32),
        ],
        scratch_shapes=[pltpu.VMEM((1, K), jnp.float32)],
    )(observation, W1, b1r, W2, b2r, cb_hi, cbt)
    return quantized, loss.reshape(())
